# baseline (device time: 131031 ns/iter reference)
import jax
import jax.numpy as jnp
from jax import lax
from jax.experimental import pallas as pl
from jax.experimental.pallas import tpu as pltpu

N_DEV = 4
SQ = 128
D = 1024
HQ = 8
HKV = 2
DH = 128
SKV = 32768
CHUNK = 512
NSTEP = SKV // CHUNK
GCOLS = (HQ // HKV) * SQ
SCALE = 0.08838834764831843


def kernel(x, Wq, Wo, K_ext, V_ext):
    x2 = x.reshape(SQ, D)
    K2 = K_ext.reshape(SKV, HKV * DH)
    V2 = V_ext.reshape(SKV, HKV * DH)

    def body(x_ref, wq_ref, wo_ref, k_ref, v_ref, out_ref,
             qs, accs, ls, ots, acc_comm, l_comm, send_sems, recv_sems):
        step = pl.program_id(0)
        my = lax.axis_index("i")

        @pl.when(step == 0)
        def _prologue():
            bar = pltpu.get_barrier_semaphore()
            for k in range(1, N_DEV):
                pl.semaphore_signal(
                    bar, inc=1,
                    device_id=(lax.rem(my + k, N_DEV),),
                    device_id_type=pl.DeviceIdType.MESH,
                )
            pl.semaphore_wait(bar, N_DEV - 1)

            qt = lax.dot_general(
                wq_ref[...].astype(jnp.bfloat16),
                x_ref[...].astype(jnp.bfloat16),
                (((0,), (1,)), ((), ())),
                preferred_element_type=jnp.float32,
            ) * SCALE
            qtb = qt.astype(jnp.bfloat16)
            for h in range(HQ):
                qs[:, h * SQ:(h + 1) * SQ] = qtb[h * DH:(h + 1) * DH, :]
            accs[...] = jnp.zeros_like(accs)
            ls[...] = jnp.zeros_like(ls)

        for kh in range(HKV):
            sl = slice(kh * GCOLS, (kh + 1) * GCOLS)
            kb = k_ref[:, kh * DH:(kh + 1) * DH].astype(jnp.bfloat16)
            vb = v_ref[:, kh * DH:(kh + 1) * DH].astype(jnp.bfloat16)
            st = lax.dot_general(
                kb, qs[:, sl], (((1,), (0,)), ((), ())),
                preferred_element_type=jnp.float32,
            )
            p = jnp.exp(st)
            ls[0:1, sl] = ls[0:1, sl] + jnp.sum(p, axis=0, keepdims=True)
            pv = lax.dot_general(
                vb, p.astype(jnp.bfloat16), (((0,), (0,)), ((), ())),
                preferred_element_type=jnp.float32,
            )
            accs[:, sl] = accs[:, sl] + pv

        @pl.when(step == NSTEP - 1)
        def _epilogue():
            for kh in range(HKV):
                sl = slice(kh * GCOLS, (kh + 1) * GCOLS)
                acc_comm[my, kh, :, :] = accs[:, sl].astype(jnp.bfloat16)
                l_comm[my, kh, 0:1, :] = ls[0:1, sl]

            sends = []
            for d in range(1, N_DEV):
                peer = lax.rem(my + d, N_DEV)
                ra = pltpu.make_async_remote_copy(
                    src_ref=acc_comm.at[my], dst_ref=acc_comm.at[my],
                    send_sem=send_sems.at[0, d - 1],
                    recv_sem=recv_sems.at[0, my],
                    device_id=(peer,), device_id_type=pl.DeviceIdType.MESH,
                )
                ra.start()
                rl = pltpu.make_async_remote_copy(
                    src_ref=l_comm.at[my], dst_ref=l_comm.at[my],
                    send_sem=send_sems.at[1, d - 1],
                    recv_sem=recv_sems.at[1, my],
                    device_id=(peer,), device_id_type=pl.DeviceIdType.MESH,
                )
                rl.start()
                sends += [ra, rl]

            for d in range(1, N_DEV):
                src = lax.rem(my + d, N_DEV)
                wa = pltpu.make_async_remote_copy(
                    src_ref=acc_comm.at[src], dst_ref=acc_comm.at[src],
                    send_sem=send_sems.at[0, 0],
                    recv_sem=recv_sems.at[0, src],
                    device_id=(my,), device_id_type=pl.DeviceIdType.MESH,
                )
                wa.wait_recv()
                wl = pltpu.make_async_remote_copy(
                    src_ref=l_comm.at[src], dst_ref=l_comm.at[src],
                    send_sem=send_sems.at[1, 0],
                    recv_sem=recv_sems.at[1, src],
                    device_id=(my,), device_id_type=pl.DeviceIdType.MESH,
                )
                wl.wait_recv()

            wo_b = wo_ref[...].astype(jnp.bfloat16)
            for kh in range(HKV):
                a_tot = jnp.zeros((DH, GCOLS), jnp.float32)
                l_tot = jnp.zeros((1, GCOLS), jnp.float32)
                for d in range(N_DEV):
                    s = lax.rem(my + d, N_DEV)
                    a_tot = a_tot + acc_comm[s, kh, :, :].astype(jnp.float32)
                    l_tot = l_tot + l_comm[s, kh, 0:1, :]
                o_kh = a_tot / l_tot
                for j in range(HQ // HKV):
                    h = kh * (HQ // HKV) + j
                    ots[h * DH:(h + 1) * DH, :] = (
                        o_kh[:, j * SQ:(j + 1) * SQ].astype(jnp.bfloat16))

            out_ref[...] = lax.dot_general(
                ots[...], wo_b, (((0,), (0,)), ((), ())),
                preferred_element_type=jnp.float32,
            )

            for r in sends:
                r.wait_send()

    out = pl.pallas_call(
        body,
        grid=(NSTEP,),
        in_specs=[
            pl.BlockSpec((SQ, D), lambda s: (0, 0)),
            pl.BlockSpec((D, D), lambda s: (0, 0)),
            pl.BlockSpec((D, D), lambda s: (0, 0)),
            pl.BlockSpec((CHUNK, HKV * DH), lambda s: (s, 0)),
            pl.BlockSpec((CHUNK, HKV * DH), lambda s: (s, 0)),
        ],
        out_specs=pl.BlockSpec((SQ, D), lambda s: (0, 0)),
        out_shape=jax.ShapeDtypeStruct((SQ, D), jnp.float32),
        scratch_shapes=[
            pltpu.VMEM((DH, HKV * GCOLS), jnp.bfloat16),
            pltpu.VMEM((DH, HKV * GCOLS), jnp.float32),
            pltpu.VMEM((8, HKV * GCOLS), jnp.float32),
            pltpu.VMEM((HQ * DH, SQ), jnp.bfloat16),
            pltpu.VMEM((N_DEV, HKV, DH, GCOLS), jnp.bfloat16),
            pltpu.VMEM((N_DEV, HKV, 8, GCOLS), jnp.float32),
            pltpu.SemaphoreType.DMA((2, N_DEV - 1)),
            pltpu.SemaphoreType.DMA((2, N_DEV)),
        ],
        compiler_params=pltpu.CompilerParams(
            collective_id=0,
            dimension_semantics=("arbitrary",),
        ),
    )(x2, Wq, Wo, K2, V2)

    return out.reshape(1, SQ, D)


# device time: 62528 ns/iter; 2.0956x vs baseline; 2.0956x over previous
import os

import jax
import jax.numpy as jnp
from jax import lax
from jax.experimental import pallas as pl
from jax.experimental.pallas import tpu as pltpu

N_DEV = 4
SQ = 128
D = 1024
HQ = 8
HKV = 2
DH = 128
SKV = 32768
CHUNK = 2048
SUBC = 2048
NSTEP = SKV // CHUNK
GCOLS = (HQ // HKV) * SQ
SCALE = 0.08838834764831843
F8 = jnp.float8_e4m3fn


def kernel(x, Wq, Wo, K_ext, V_ext):
    x2 = x.reshape(SQ, D)
    K3 = K_ext.reshape(SKV, HKV, DH)
    V3 = V_ext.reshape(SKV, HKV, DH)

    def body(x_ref, wq_ref, wo_ref, k_ref, v_ref, out_ref,
             qs, accs, ls, ots, kbuf, vbuf, copy_sems,
             acc_comm, l_comm, send_sems, recv_sems):
        step = pl.program_id(0)
        my = lax.axis_index("i")

        k2 = k_ref.reshape(SKV, HKV * DH)
        v2 = v_ref.reshape(SKV, HKV * DH)

        def fetch(chunk, slot):
            pltpu.make_async_copy(
                k2.at[pl.ds(chunk * CHUNK, CHUNK), :],
                kbuf.at[slot],
                copy_sems.at[slot, 0],
            ).start()
            pltpu.make_async_copy(
                v2.at[pl.ds(chunk * CHUNK, CHUNK), :],
                vbuf.at[slot],
                copy_sems.at[slot, 1],
            ).start()

        def wait_slot(slot):
            for t, buf in enumerate((kbuf, vbuf)):
                pltpu.make_async_copy(
                    k2.at[pl.ds(0, CHUNK), :],
                    buf.at[slot],
                    copy_sems.at[slot, t],
                ).wait()

        _abl = os.environ.get("SCB_ABLATE", "")

        if "nodma" not in _abl:
            @pl.when(step == 0)
            def _start_pipe():
                fetch(0, 0)
                fetch(1, 1)

        @pl.when(step == 0)
        def _prologue():
            bar = pltpu.get_barrier_semaphore()
            for k in range(1, N_DEV):
                pl.semaphore_signal(
                    bar, inc=1,
                    device_id=(lax.rem(my + k, N_DEV),),
                    device_id_type=pl.DeviceIdType.MESH,
                )
            pl.semaphore_wait(bar, N_DEV - 1)

            qt = lax.dot_general(
                wq_ref[...].astype(jnp.bfloat16),
                x_ref[...].astype(jnp.bfloat16),
                (((0,), (1,)), ((), ())),
                preferred_element_type=jnp.float32,
            ) * SCALE
            qtb = qt.astype(jnp.bfloat16)
            qs[...] = jnp.zeros_like(qs)
            for h in range(HQ):
                kh = h // (HQ // HKV)
                qs[kh * DH:(kh + 1) * DH, h * SQ:(h + 1) * SQ] = (
                    qtb[h * DH:(h + 1) * DH, :])
            accs[...] = jnp.zeros_like(accs)
            ls[...] = jnp.zeros_like(ls)

        slot = lax.rem(step, 2)
        if "nodma" not in _abl:
            wait_slot(slot)
        if "dmaonly" in _abl:
            accs[0:8, 0:128] = (accs[0:8, 0:128]
                                + kbuf[slot, 0:8, 0:128]
                                + vbuf[slot, 0:8, 0:128])
        else:
            for sub in range(CHUNK // SUBC):
                rows = slice(sub * SUBC, (sub + 1) * SUBC)
                kb = kbuf[slot, rows, :].astype(jnp.bfloat16)
                vb = vbuf[slot, rows, :].astype(jnp.bfloat16)
                st = lax.dot_general(
                    kb, qs[...], (((1,), (0,)), ((), ())),
                    preferred_element_type=jnp.float32,
                )
                p = st if "noexp" in _abl else jnp.exp(st)
                pb = p.astype(jnp.bfloat16)
                ls[0:1, :] = ls[0:1, :] + jnp.sum(
                    pb, axis=0, keepdims=True, dtype=jnp.float32)
                pv = lax.dot_general(
                    vb, pb, (((0,), (0,)), ((), ())),
                    preferred_element_type=jnp.float32,
                )
                for kh in range(HKV):
                    sl = slice(kh * GCOLS, (kh + 1) * GCOLS)
                    accs[:, sl] = (accs[:, sl]
                                   + pv[kh * DH:(kh + 1) * DH, sl])

        if "nodma" not in _abl:
            @pl.when(step + 2 < NSTEP)
            def _next_fetch():
                fetch(step + 2, slot)

        @pl.when(step == NSTEP - 1)
        def _epilogue():
            for kh in range(HKV):
                sl = slice(kh * GCOLS, (kh + 1) * GCOLS)
                acc_comm[my, kh, :, :] = accs[:, sl].astype(jnp.bfloat16)
                l_comm[my, kh, 0:1, :] = ls[0:1, sl]

            sends = []
            for d in range(1, N_DEV):
                peer = lax.rem(my + d, N_DEV)
                ra = pltpu.make_async_remote_copy(
                    src_ref=acc_comm.at[my], dst_ref=acc_comm.at[my],
                    send_sem=send_sems.at[0, d - 1],
                    recv_sem=recv_sems.at[0, my],
                    device_id=(peer,), device_id_type=pl.DeviceIdType.MESH,
                )
                ra.start()
                rl = pltpu.make_async_remote_copy(
                    src_ref=l_comm.at[my], dst_ref=l_comm.at[my],
                    send_sem=send_sems.at[1, d - 1],
                    recv_sem=recv_sems.at[1, my],
                    device_id=(peer,), device_id_type=pl.DeviceIdType.MESH,
                )
                rl.start()
                sends += [ra, rl]

            for d in range(1, N_DEV):
                src = lax.rem(my + d, N_DEV)
                wa = pltpu.make_async_remote_copy(
                    src_ref=acc_comm.at[src], dst_ref=acc_comm.at[src],
                    send_sem=send_sems.at[0, 0],
                    recv_sem=recv_sems.at[0, src],
                    device_id=(my,), device_id_type=pl.DeviceIdType.MESH,
                )
                wa.wait_recv()
                wl = pltpu.make_async_remote_copy(
                    src_ref=l_comm.at[src], dst_ref=l_comm.at[src],
                    send_sem=send_sems.at[1, 0],
                    recv_sem=recv_sems.at[1, src],
                    device_id=(my,), device_id_type=pl.DeviceIdType.MESH,
                )
                wl.wait_recv()

            wo_b = wo_ref[...].astype(jnp.bfloat16)
            for kh in range(HKV):
                a_tot = jnp.zeros((DH, GCOLS), jnp.float32)
                l_tot = jnp.zeros((1, GCOLS), jnp.float32)
                for d in range(N_DEV):
                    s = lax.rem(my + d, N_DEV)
                    a_tot = a_tot + acc_comm[s, kh, :, :].astype(jnp.float32)
                    l_tot = l_tot + l_comm[s, kh, 0:1, :]
                o_kh = a_tot / l_tot
                for j in range(HQ // HKV):
                    h = kh * (HQ // HKV) + j
                    ots[h * DH:(h + 1) * DH, :] = (
                        o_kh[:, j * SQ:(j + 1) * SQ].astype(jnp.bfloat16))

            out_ref[...] = lax.dot_general(
                ots[...], wo_b, (((0,), (0,)), ((), ())),
                preferred_element_type=jnp.float32,
            )

            for r in sends:
                r.wait_send()

    out = pl.pallas_call(
        body,
        grid=(NSTEP,),
        in_specs=[
            pl.BlockSpec((SQ, D), lambda s: (0, 0)),
            pl.BlockSpec((D, D), lambda s: (0, 0)),
            pl.BlockSpec((D, D), lambda s: (0, 0)),
            pl.BlockSpec(memory_space=pl.ANY),
            pl.BlockSpec(memory_space=pl.ANY),
        ],
        out_specs=pl.BlockSpec((SQ, D), lambda s: (0, 0)),
        out_shape=jax.ShapeDtypeStruct((SQ, D), jnp.float32),
        scratch_shapes=[
            pltpu.VMEM((HKV * DH, HKV * GCOLS), jnp.bfloat16),
            pltpu.VMEM((DH, HKV * GCOLS), jnp.float32),
            pltpu.VMEM((8, HKV * GCOLS), jnp.float32),
            pltpu.VMEM((HQ * DH, SQ), jnp.bfloat16),
            pltpu.VMEM((2, CHUNK, HKV * DH), jnp.float32),
            pltpu.VMEM((2, CHUNK, HKV * DH), jnp.float32),
            pltpu.SemaphoreType.DMA((2, 2)),
            pltpu.VMEM((N_DEV, HKV, DH, GCOLS), jnp.bfloat16),
            pltpu.VMEM((N_DEV, HKV, 8, GCOLS), jnp.float32),
            pltpu.SemaphoreType.DMA((2, N_DEV - 1)),
            pltpu.SemaphoreType.DMA((2, N_DEV)),
        ],
        compiler_params=pltpu.CompilerParams(
            collective_id=0,
            dimension_semantics=("arbitrary",),
        ),
    )(x2, Wq, Wo, K3, V3)

    return out.reshape(1, SQ, D)
